# baseline (speedup 1.0000x reference)
import functools

import jax
import jax.numpy as jnp
from jax import lax
from jax.experimental import pallas as pl
from jax.experimental.pallas import tpu as pltpu
from jax.experimental.pallas import tpu_sc as plsc

NC = 2
NS = 16
NW = NC * NS
LANES = 16
CHUNK = 128
N_PASS = 2
TBLK = 512


def _mesh():
    return plsc.VectorSubcoreMesh(
        core_axis_name="c", subcore_axis_name="s",
        num_cores=NC, num_subcores=NS,
    )


def _cparams():
    return pltpu.CompilerParams(needs_layout_passes=False)


@functools.lru_cache(maxsize=None)
def _build_transpose(V: int, D: int):
    pairs = 2 * D

    def body(e_ref, x_ref, o_ref):
        t = lax.dot_general(
            e_ref[...], x_ref[...], (((0,), (1,)), ((), ())),
            preferred_element_type=jnp.float32,
            precision=lax.Precision.HIGHEST)
        half = TBLK // 2
        o_ref[...] = jnp.concatenate([t[:half], t[half:]], axis=1)

    call = pl.pallas_call(
        body,
        grid=((V + TBLK - 1) // TBLK,),
        in_specs=[pl.BlockSpec((TBLK, TBLK), lambda i: (0, 0)),
                  pl.BlockSpec((D, TBLK), lambda i: (0, i))],
        out_specs=pl.BlockSpec((TBLK // 2, pairs), lambda i: (i,  0)),
        out_shape=jax.ShapeDtypeStruct((V // 2, pairs), jnp.float32),
    )

    def run(table_t):
        v = jnp.arange(TBLK)[:, None]
        w = jnp.arange(TBLK)[None, :]
        half = TBLK // 2
        target = jnp.where(w < half, 2 * w, 2 * (w - half) + 1)
        e = (v == target).astype(jnp.float32)
        return call(e, table_t).reshape(-1)

    return run


def _issue_rows(idx_v, table_hbm, dst_v, sem, b, rows_per_pass, D):
    r0 = b * LANES
    ti = idx_v[pl.ds(r0, LANES)]
    dr0 = (b % (rows_per_pass // LANES)) * LANES
    for r in range(LANES):
        pltpu.async_copy(table_hbm.at[pl.ds(ti[r] * D, D)],
                         dst_v.at[pl.ds((dr0 + r) * D, D)], sem)


@functools.lru_cache(maxsize=None)
def _build_k1(B: int, D: int):
    b_per_w = B // NW
    n_chunks = b_per_w // CHUNK
    rows_per_pass = b_per_w // N_PASS

    @functools.partial(
        pl.kernel,
        out_type=(jax.ShapeDtypeStruct((B * D,), jnp.float32),
                  jax.ShapeDtypeStruct((B,), jnp.float32)),
        mesh=_mesh(),
        compiler_params=_cparams(),
        scratch_types=[
            pltpu.VMEM((b_per_w,), jnp.int32),
            pltpu.VMEM((rows_per_pass * D,), jnp.float32),
            pltpu.VMEM((b_per_w,), jnp.float32),
            [pltpu.SemaphoreType.DMA] * 2,
        ],
    )
    def k1(tok_hbm, temb_hbm, tb_hbm, rows_out, bias_out,
           idx_i, wi_v, bi_v, sems):
        wid = lax.axis_index("s") * NC + lax.axis_index("c")
        base = wid * b_per_w
        pltpu.sync_copy(tok_hbm.at[pl.ds(base, b_per_w)], idx_i)
        for c in range(n_chunks):
            sl = pl.ds(c * CHUNK, CHUNK)
            pltpu.async_copy(tb_hbm.at[idx_i.at[sl]], bi_v.at[sl], sems[1])

        def issue(b, carry):
            _issue_rows(idx_i, temb_hbm, wi_v, sems[0], b, rows_per_pass, D)
            return carry

        blocks_per_pass = rows_per_pass // LANES
        for p in range(N_PASS):
            lax.fori_loop(p * blocks_per_pass, (p + 1) * blocks_per_pass,
                          issue, 0)
            pltpu.make_async_copy(temb_hbm.at[pl.ds(0, rows_per_pass * D)],
                                  wi_v, sems[0]).wait()
            pltpu.sync_copy(
                wi_v,
                rows_out.at[pl.ds((base + p * rows_per_pass) * D,
                                  rows_per_pass * D)])
        pltpu.make_async_copy(tb_hbm.at[pl.ds(0, b_per_w)], bi_v,
                              sems[1]).wait()
        pltpu.sync_copy(bi_v, bias_out.at[pl.ds(base, b_per_w)])

    return k1


@functools.lru_cache(maxsize=None)
def _build_k2(B: int, D: int):
    b_per_w = B // NW
    n_chunks = b_per_w // CHUNK
    n_seg = D // LANES
    rows_per_pass = b_per_w // N_PASS

    @functools.partial(
        pl.kernel,
        out_type=jax.ShapeDtypeStruct((B,), jnp.float32),
        mesh=_mesh(),
        compiler_params=_cparams(),
        scratch_types=[
            pltpu.VMEM((b_per_w,), jnp.int32),
            pltpu.VMEM((rows_per_pass * D,), jnp.float32),
            pltpu.VMEM((rows_per_pass * D,), jnp.float32),
            pltpu.VMEM((b_per_w,), jnp.float32),
            pltpu.VMEM((b_per_w,), jnp.float32),
            pltpu.VMEM((b_per_w,), jnp.float32),
            [pltpu.SemaphoreType.DMA] * 3,
        ],
    )
    def k2(ctx_hbm, cemb_hbm, cb_hbm, wi_hbm, bi_hbm, out_hbm,
           idx_j, wi_v, wj_v, bi_v, bj_v, out_v, sems):
        wid = lax.axis_index("s") * NC + lax.axis_index("c")
        base = wid * b_per_w
        pltpu.sync_copy(ctx_hbm.at[pl.ds(base, b_per_w)], idx_j)
        for c in range(n_chunks):
            sl = pl.ds(c * CHUNK, CHUNK)
            pltpu.async_copy(cb_hbm.at[idx_j.at[sl]], bj_v.at[sl], sems[1])
        pltpu.sync_copy(bi_hbm.at[pl.ds(base, b_per_w)], bi_v)

        def issue(b, carry):
            _issue_rows(idx_j, cemb_hbm, wj_v, sems[0], b, rows_per_pass, D)
            return carry

        lane_ids = lax.iota(jnp.int32, LANES)

        def block(b, carry):
            r0 = b * LANES
            dr0 = (b % (rows_per_pass // LANES)) * LANES
            sums = jnp.zeros((LANES,), jnp.float32)
            for r in range(LANES):
                row = dr0 + r
                acc = (wi_v[pl.ds(row * D, LANES)]
                       * wj_v[pl.ds(row * D, LANES)])
                for s in range(1, n_seg):
                    sl = pl.ds(row * D + s * LANES, LANES)
                    acc = acc + wi_v[sl] * wj_v[sl]
                sums = jnp.where(lane_ids == r, jnp.sum(acc), sums)
            blk = pl.ds(r0, LANES)
            out_v[blk] = sums + bi_v[blk] + bj_v[blk]
            return carry

        pltpu.make_async_copy(cb_hbm.at[pl.ds(0, b_per_w)], bj_v,
                              sems[1]).wait()
        blocks_per_pass = rows_per_pass // LANES
        for p in range(N_PASS):
            lax.fori_loop(p * blocks_per_pass, (p + 1) * blocks_per_pass,
                          issue, 0)
            pltpu.async_copy(
                wi_hbm.at[pl.ds((base + p * rows_per_pass) * D,
                                rows_per_pass * D)],
                wi_v, sems[2])
            pltpu.make_async_copy(
                wi_hbm.at[pl.ds(0, rows_per_pass * D)], wi_v,
                sems[2]).wait()
            pltpu.make_async_copy(cemb_hbm.at[pl.ds(0, rows_per_pass * D)],
                                  wj_v, sems[0]).wait()
            lax.fori_loop(p * blocks_per_pass, (p + 1) * blocks_per_pass,
                          block, 0)
        pltpu.sync_copy(out_v, out_hbm.at[pl.ds(base, b_per_w)])

    return k2


def kernel(token, context_token, token_embedding, context_embedding,
           token_bias, context_bias):
    B = token.shape[0]
    V, D = token_embedding.shape
    tok = token.astype(jnp.int32)
    ctx = context_token.astype(jnp.int32)
    tb = token_bias.sum(axis=1)
    cb = context_bias.sum(axis=1)
    transpose = _build_transpose(V, D)
    t1 = transpose(token_embedding.T)
    t2 = transpose(context_embedding.T)
    wi_rows, bi = _build_k1(B, D)(tok, t1, tb)
    return _build_k2(B, D)(ctx, t2, cb, wi_rows, bi)


# speedup vs baseline: 2.1507x; 2.1507x over previous
import functools

import jax
import jax.numpy as jnp
from jax import lax
from jax.experimental import pallas as pl
from jax.experimental.pallas import tpu as pltpu
from jax.experimental.pallas import tpu_sc as plsc

NC = 2
NS = 16
NW = NC * NS
LANES = 16
CHUNK = 128
N_PASS = 2
VCH = 512


def _mesh():
    return plsc.VectorSubcoreMesh(
        core_axis_name="c", subcore_axis_name="s",
        num_cores=NC, num_subcores=NS,
    )


def _cparams():
    return pltpu.CompilerParams(
        needs_layout_passes=False, use_tc_tiling_on_sc=True)


@functools.lru_cache(maxsize=None)
def _build_sc_transpose(V: int, D: int):
    full = V // VCH
    tail = V % VCH
    max_c = full + (1 if tail else 0)
    chunks_per_w = (max_c + NW - 1) // NW
    iota = None

    @functools.partial(
        pl.kernel,
        out_type=jax.ShapeDtypeStruct((V * D,), jnp.float32),
        mesh=_mesh(),
        compiler_params=_cparams(),
        scratch_types=[
            pltpu.VMEM((D, VCH), jnp.float32),
            pltpu.VMEM((VCH * D,), jnp.float32),
            pltpu.VMEM((D, tail), jnp.float32),
            pltpu.VMEM((tail * D,), jnp.float32),
        ],
    )
    def sct(tbl_t, tail_t, flat, buf, obuf, tbuf, tobuf):
        wid = lax.axis_index("s") * NC + lax.axis_index("c")
        d_ids = lax.iota(jnp.int32, LANES)

        def transpose_block(src, dst, v, carry):
            for r in range(LANES):
                vv = jnp.full((LANES,), v * LANES + r, jnp.int32)
                for d0 in range(0, D, LANES):
                    g = plsc.load_gather(src, [d0 + d_ids, vv])
                    dst[pl.ds((v * LANES + r) * D + d0, LANES)] = g
            return carry

        for j in range(chunks_per_w):
            c = wid + NW * j

            @pl.when(c < full)
            def _():
                v0 = c * VCH
                pltpu.sync_copy(tbl_t.at[:, pl.ds(v0, VCH)], buf)
                lax.fori_loop(
                    0, VCH // LANES,
                    functools.partial(transpose_block, buf, obuf), 0)
                pltpu.sync_copy(obuf, flat.at[pl.ds(v0 * D, VCH * D)])

        if tail:
            @pl.when(wid == NW - 1)
            def _():
                pltpu.sync_copy(tail_t, tbuf)
                lax.fori_loop(
                    0, tail // LANES,
                    functools.partial(transpose_block, tbuf, tobuf), 0)
                pltpu.sync_copy(
                    tobuf, flat.at[pl.ds(full * VCH * D, tail * D)])

    return sct


def _issue_rows_2d(idx_v, table_hbm, dst_v, sem, b, rows_per_pass):
    r0 = b * LANES
    ti = idx_v[pl.ds(r0, LANES)]
    dr0 = (b % (rows_per_pass // LANES)) * LANES
    for r in range(LANES):
        pltpu.async_copy(table_hbm.at[pl.ds(ti[r], 1)],
                         dst_v.at[pl.ds(dr0 + r, 1)], sem)


def _issue_rows_flat(idx_v, flat_hbm, dst_v, sem, b, rows_per_pass, D):
    r0 = b * LANES
    ti = idx_v[pl.ds(r0, LANES)]
    dr0 = (b % (rows_per_pass // LANES)) * LANES
    for r in range(LANES):
        src = pl.ds(pl.multiple_of(ti[r] * D, D), D)
        pltpu.async_copy(flat_hbm.at[src],
                         dst_v.at[pl.ds((dr0 + r) * D, D)], sem)


@functools.lru_cache(maxsize=None)
def _build_k1(B: int, D: int):
    b_per_w = B // NW
    n_chunks = b_per_w // CHUNK
    rows_per_pass = b_per_w // N_PASS

    @functools.partial(
        pl.kernel,
        out_type=(jax.ShapeDtypeStruct((B, D), jnp.float32),
                  jax.ShapeDtypeStruct((B,), jnp.float32)),
        mesh=_mesh(),
        compiler_params=_cparams(),
        scratch_types=[
            pltpu.VMEM((b_per_w,), jnp.int32),
            pltpu.VMEM((rows_per_pass, D), jnp.float32),
            pltpu.VMEM((b_per_w,), jnp.float32),
            [pltpu.SemaphoreType.DMA] * 2,
        ],
    )
    def k1(tok_hbm, temb_hbm, tb_hbm, rows_out, bias_out,
           idx_i, wi_v, bi_v, sems):
        wid = lax.axis_index("s") * NC + lax.axis_index("c")
        base = wid * b_per_w
        pltpu.sync_copy(tok_hbm.at[pl.ds(base, b_per_w)], idx_i)
        for c in range(n_chunks):
            sl = pl.ds(c * CHUNK, CHUNK)
            pltpu.async_copy(tb_hbm.at[idx_i.at[sl]], bi_v.at[sl], sems[1])

        def issue(b, carry):
            _issue_rows_2d(idx_i, temb_hbm, wi_v, sems[0], b, rows_per_pass)
            return carry

        blocks_per_pass = rows_per_pass // LANES
        for p in range(N_PASS):
            lax.fori_loop(p * blocks_per_pass, (p + 1) * blocks_per_pass,
                          issue, 0)
            pltpu.make_async_copy(temb_hbm.at[pl.ds(0, rows_per_pass)],
                                  wi_v, sems[0]).wait()
            pltpu.sync_copy(
                wi_v,
                rows_out.at[pl.ds(base + p * rows_per_pass,
                                  rows_per_pass)])
        pltpu.make_async_copy(tb_hbm.at[pl.ds(0, b_per_w)], bi_v,
                              sems[1]).wait()
        pltpu.sync_copy(bi_v, bias_out.at[pl.ds(base, b_per_w)])

    return k1


@functools.lru_cache(maxsize=None)
def _build_k2(B: int, D: int):
    b_per_w = B // NW
    n_chunks = b_per_w // CHUNK
    n_seg = D // LANES
    rows_per_pass = b_per_w // N_PASS

    @functools.partial(
        pl.kernel,
        out_type=jax.ShapeDtypeStruct((B,), jnp.float32),
        mesh=_mesh(),
        compiler_params=_cparams(),
        scratch_types=[
            pltpu.VMEM((b_per_w,), jnp.int32),
            pltpu.VMEM((rows_per_pass, D), jnp.float32),
            pltpu.VMEM((rows_per_pass * D,), jnp.float32),
            pltpu.VMEM((b_per_w,), jnp.float32),
            pltpu.VMEM((b_per_w,), jnp.float32),
            pltpu.VMEM((b_per_w,), jnp.float32),
            [pltpu.SemaphoreType.DMA] * 3,
        ],
    )
    def k2(ctx_hbm, cflat_hbm, cb_hbm, wi_hbm, bi_hbm, out_hbm,
           idx_j, wi_v, wj_v, bi_v, bj_v, out_v, sems):
        wid = lax.axis_index("s") * NC + lax.axis_index("c")
        base = wid * b_per_w
        pltpu.sync_copy(ctx_hbm.at[pl.ds(base, b_per_w)], idx_j)
        for c in range(n_chunks):
            sl = pl.ds(c * CHUNK, CHUNK)
            pltpu.async_copy(cb_hbm.at[idx_j.at[sl]], bj_v.at[sl], sems[1])
        pltpu.sync_copy(bi_hbm.at[pl.ds(base, b_per_w)], bi_v)

        def issue(b, carry):
            _issue_rows_flat(idx_j, cflat_hbm, wj_v, sems[0], b,
                             rows_per_pass, D)
            return carry

        lane_ids = lax.iota(jnp.int32, LANES)

        def block(b, carry):
            r0 = b * LANES
            dr0 = (b % (rows_per_pass // LANES)) * LANES
            sums = jnp.zeros((LANES,), jnp.float32)
            for r in range(LANES):
                row = dr0 + r
                acc = (wi_v[row, pl.ds(0, LANES)]
                       * wj_v[pl.ds(row * D, LANES)])
                for s in range(1, n_seg):
                    acc = acc + (wi_v[row, pl.ds(s * LANES, LANES)]
                                 * wj_v[pl.ds(row * D + s * LANES, LANES)])
                sums = jnp.where(lane_ids == r, jnp.sum(acc), sums)
            blk = pl.ds(r0, LANES)
            out_v[blk] = sums + bi_v[blk] + bj_v[blk]
            return carry

        pltpu.make_async_copy(cb_hbm.at[pl.ds(0, b_per_w)], bj_v,
                              sems[1]).wait()
        blocks_per_pass = rows_per_pass // LANES
        for p in range(N_PASS):
            lax.fori_loop(p * blocks_per_pass, (p + 1) * blocks_per_pass,
                          issue, 0)
            pltpu.async_copy(
                wi_hbm.at[pl.ds(base + p * rows_per_pass, rows_per_pass)],
                wi_v, sems[2])
            pltpu.make_async_copy(
                wi_hbm.at[pl.ds(0, rows_per_pass)], wi_v, sems[2]).wait()
            pltpu.make_async_copy(
                cflat_hbm.at[pl.ds(0, rows_per_pass * D)], wj_v,
                sems[0]).wait()
            lax.fori_loop(p * blocks_per_pass, (p + 1) * blocks_per_pass,
                          block, 0)
        pltpu.sync_copy(out_v, out_hbm.at[pl.ds(base, b_per_w)])

    return k2


def kernel(token, context_token, token_embedding, context_embedding,
           token_bias, context_bias):
    B = token.shape[0]
    V, D = token_embedding.shape
    tok = token.astype(jnp.int32)
    ctx = context_token.astype(jnp.int32)
    tb = token_bias.sum(axis=1)
    cb = context_bias.sum(axis=1)
    cemb_t = context_embedding.T
    tail = V % VCH
    tail_t = cemb_t[:, V - tail:]
    cflat = _build_sc_transpose(V, D)(cemb_t, tail_t)
    wi_rows, bi = _build_k1(B, D)(tok, token_embedding, tb)
    return _build_k2(B, D)(ctx, cflat, cb, wi_rows, bi)


# speedup vs baseline: 4.9929x; 2.3215x over previous
import functools

import jax
import jax.numpy as jnp
from jax import lax
from jax.experimental import pallas as pl
from jax.experimental.pallas import tpu as pltpu
from jax.experimental.pallas import tpu_sc as plsc

NC = 2
NS = 16
NW = NC * NS
LANES = 16
CHUNK = 128
N_PASS = 2


def _mesh():
    return plsc.VectorSubcoreMesh(
        core_axis_name="c", subcore_axis_name="s",
        num_cores=NC, num_subcores=NS,
    )


def _cparams():
    return pltpu.CompilerParams(
        needs_layout_passes=False, use_tc_tiling_on_sc=True)


def _issue_rows(idx_v, table_hbm, dst_v, sem, b, rows_per_pass):
    r0 = b * LANES
    ti = idx_v[pl.ds(r0, LANES)]
    dr0 = (b % (rows_per_pass // LANES)) * LANES
    for r in range(LANES):
        pltpu.async_copy(table_hbm.at[pl.ds(ti[r], 1)],
                         dst_v.at[pl.ds(dr0 + r, 1)], sem)


@functools.lru_cache(maxsize=None)
def _build_k1(B: int, D: int):
    b_per_w = B // NW
    n_chunks = b_per_w // CHUNK
    rows_per_pass = b_per_w // N_PASS

    @functools.partial(
        pl.kernel,
        out_type=(jax.ShapeDtypeStruct((B, D), jnp.float32),
                  jax.ShapeDtypeStruct((B,), jnp.float32)),
        mesh=_mesh(),
        compiler_params=_cparams(),
        scratch_types=[
            pltpu.VMEM((b_per_w,), jnp.int32),
            pltpu.VMEM((rows_per_pass, D), jnp.float32),
            pltpu.VMEM((b_per_w,), jnp.float32),
            [pltpu.SemaphoreType.DMA] * 2,
        ],
    )
    def k1(tok_hbm, temb_hbm, tb_hbm, rows_out, bias_out,
           idx_i, wi_v, bi_v, sems):
        wid = lax.axis_index("s") * NC + lax.axis_index("c")
        base = wid * b_per_w
        pltpu.sync_copy(tok_hbm.at[pl.ds(base, b_per_w)], idx_i)
        for c in range(n_chunks):
            sl = pl.ds(c * CHUNK, CHUNK)
            pltpu.async_copy(tb_hbm.at[idx_i.at[sl]], bi_v.at[sl], sems[1])

        def issue(b, carry):
            _issue_rows(idx_i, temb_hbm, wi_v, sems[0], b, rows_per_pass)
            return carry

        blocks_per_pass = rows_per_pass // LANES
        for p in range(N_PASS):
            lax.fori_loop(p * blocks_per_pass, (p + 1) * blocks_per_pass,
                          issue, 0)
            pltpu.make_async_copy(temb_hbm.at[pl.ds(0, rows_per_pass)],
                                  wi_v, sems[0]).wait()
            pltpu.sync_copy(
                wi_v,
                rows_out.at[pl.ds(base + p * rows_per_pass,
                                  rows_per_pass)])
        pltpu.make_async_copy(tb_hbm.at[pl.ds(0, b_per_w)], bi_v,
                              sems[1]).wait()
        pltpu.sync_copy(bi_v, bias_out.at[pl.ds(base, b_per_w)])

    return k1


@functools.lru_cache(maxsize=None)
def _build_k2(B: int, D: int):
    b_per_w = B // NW
    n_chunks = b_per_w // CHUNK
    n_seg = D // LANES
    rows_per_pass = b_per_w // N_PASS

    @functools.partial(
        pl.kernel,
        out_type=jax.ShapeDtypeStruct((B,), jnp.float32),
        mesh=_mesh(),
        compiler_params=_cparams(),
        scratch_types=[
            pltpu.VMEM((b_per_w,), jnp.int32),
            pltpu.VMEM((rows_per_pass, D), jnp.float32),
            pltpu.VMEM((rows_per_pass, D), jnp.float32),
            pltpu.VMEM((b_per_w,), jnp.float32),
            pltpu.VMEM((b_per_w,), jnp.float32),
            pltpu.VMEM((b_per_w,), jnp.float32),
            [pltpu.SemaphoreType.DMA] * 3,
        ],
    )
    def k2(ctx_hbm, cemb_hbm, cb_hbm, wi_hbm, bi_hbm, out_hbm,
           idx_j, wi_v, wj_v, bi_v, bj_v, out_v, sems):
        wid = lax.axis_index("s") * NC + lax.axis_index("c")
        base = wid * b_per_w
        pltpu.sync_copy(ctx_hbm.at[pl.ds(base, b_per_w)], idx_j)
        for c in range(n_chunks):
            sl = pl.ds(c * CHUNK, CHUNK)
            pltpu.async_copy(cb_hbm.at[idx_j.at[sl]], bj_v.at[sl], sems[1])
        pltpu.sync_copy(bi_hbm.at[pl.ds(base, b_per_w)], bi_v)

        def issue(b, carry):
            _issue_rows(idx_j, cemb_hbm, wj_v, sems[0], b, rows_per_pass)
            return carry

        lane_ids = lax.iota(jnp.int32, LANES)

        def block(b, carry):
            r0 = b * LANES
            dr0 = (b % (rows_per_pass // LANES)) * LANES
            sums = jnp.zeros((LANES,), jnp.float32)
            for r in range(LANES):
                row = dr0 + r
                acc = wi_v[row, pl.ds(0, LANES)] * wj_v[row, pl.ds(0, LANES)]
                for s in range(1, n_seg):
                    sl = pl.ds(s * LANES, LANES)
                    acc = acc + wi_v[row, sl] * wj_v[row, sl]
                sums = jnp.where(lane_ids == r, jnp.sum(acc), sums)
            blk = pl.ds(r0, LANES)
            out_v[blk] = sums + bi_v[blk] + bj_v[blk]
            return carry

        pltpu.make_async_copy(cb_hbm.at[pl.ds(0, b_per_w)], bj_v,
                              sems[1]).wait()
        blocks_per_pass = rows_per_pass // LANES
        for p in range(N_PASS):
            pltpu.async_copy(
                wi_hbm.at[pl.ds(base + p * rows_per_pass, rows_per_pass)],
                wi_v, sems[2])
            lax.fori_loop(p * blocks_per_pass, (p + 1) * blocks_per_pass,
                          issue, 0)
            pltpu.make_async_copy(
                wi_hbm.at[pl.ds(0, rows_per_pass)], wi_v, sems[2]).wait()
            pltpu.make_async_copy(cemb_hbm.at[pl.ds(0, rows_per_pass)],
                                  wj_v, sems[0]).wait()
            lax.fori_loop(p * blocks_per_pass, (p + 1) * blocks_per_pass,
                          block, 0)
        pltpu.sync_copy(out_v, out_hbm.at[pl.ds(base, b_per_w)])

    return k2


def kernel(token, context_token, token_embedding, context_embedding,
           token_bias, context_bias):
    B = token.shape[0]
    D = token_embedding.shape[1]
    tok = token.astype(jnp.int32)
    ctx = context_token.astype(jnp.int32)
    tb = token_bias.sum(axis=1)
    cb = context_bias.sum(axis=1)
    wi_rows, bi = _build_k1(B, D)(tok, token_embedding, tb)
    return _build_k2(B, D)(ctx, context_embedding, cb, wi_rows, bi)


# speedup vs baseline: 5.0369x; 1.0088x over previous
import functools

import jax
import jax.numpy as jnp
from jax import lax
from jax.experimental import pallas as pl
from jax.experimental.pallas import tpu as pltpu
from jax.experimental.pallas import tpu_sc as plsc

NC = 2
NS = 16
NW = NC * NS
LANES = 16
CHUNK = 128
N_PASS = 2


def _mesh():
    return plsc.VectorSubcoreMesh(
        core_axis_name="c", subcore_axis_name="s",
        num_cores=NC, num_subcores=NS,
    )


def _cparams():
    return pltpu.CompilerParams(
        needs_layout_passes=False, use_tc_tiling_on_sc=True)


def _issue_rows(idx_v, table_hbm, dst_v, sem, b, rows_per_pass):
    r0 = b * LANES
    ti = idx_v[pl.ds(r0, LANES)]
    dr0 = (b % (rows_per_pass // LANES)) * LANES
    for r in range(LANES):
        pltpu.async_copy(table_hbm.at[pl.ds(ti[r], 1)],
                         dst_v.at[pl.ds(dr0 + r, 1)], sem)


@functools.lru_cache(maxsize=None)
def _build_k1(B: int, D: int):
    b_per_w = B // NW
    n_chunks = b_per_w // CHUNK
    rows_per_pass = b_per_w // N_PASS

    @functools.partial(
        pl.kernel,
        out_type=(jax.ShapeDtypeStruct((B, D), jnp.float32),
                  jax.ShapeDtypeStruct((B,), jnp.float32)),
        mesh=_mesh(),
        compiler_params=_cparams(),
        scratch_types=[
            pltpu.VMEM((b_per_w,), jnp.int32),
            pltpu.VMEM((rows_per_pass, D), jnp.float32),
            pltpu.VMEM((b_per_w,), jnp.float32),
            [pltpu.SemaphoreType.DMA] * 2,
        ],
    )
    def k1(tok_hbm, temb_hbm, tb_hbm, rows_out, bias_out,
           idx_i, wi_v, bi_v, sems):
        wid = lax.axis_index("s") * NC + lax.axis_index("c")
        base = wid * b_per_w
        pltpu.sync_copy(tok_hbm.at[pl.ds(base, b_per_w)], idx_i)
        for c in range(n_chunks):
            sl = pl.ds(c * CHUNK, CHUNK)
            pltpu.async_copy(tb_hbm.at[idx_i.at[sl]], bi_v.at[sl], sems[1])

        def issue(b, carry):
            _issue_rows(idx_i, temb_hbm, wi_v, sems[0], b, rows_per_pass)
            return carry

        blocks_per_pass = rows_per_pass // LANES
        for p in range(N_PASS):
            lax.fori_loop(p * blocks_per_pass, (p + 1) * blocks_per_pass,
                          issue, 0)
            pltpu.make_async_copy(temb_hbm.at[pl.ds(0, rows_per_pass)],
                                  wi_v, sems[0]).wait()
            pltpu.sync_copy(
                wi_v,
                rows_out.at[pl.ds(base + p * rows_per_pass,
                                  rows_per_pass)])
        pltpu.make_async_copy(tb_hbm.at[pl.ds(0, b_per_w)], bi_v,
                              sems[1]).wait()
        pltpu.sync_copy(bi_v, bias_out.at[pl.ds(base, b_per_w)])

    return k1


@functools.lru_cache(maxsize=None)
def _build_k2(B: int, D: int):
    b_per_w = B // NW
    n_chunks = b_per_w // CHUNK
    n_seg = D // LANES
    rows_per_pass = b_per_w // N_PASS

    @functools.partial(
        pl.kernel,
        out_type=jax.ShapeDtypeStruct((B,), jnp.float32),
        mesh=_mesh(),
        compiler_params=_cparams(),
        scratch_types=[
            pltpu.VMEM((b_per_w,), jnp.int32),
            pltpu.VMEM((rows_per_pass, D), jnp.float32),
            pltpu.VMEM((rows_per_pass, D), jnp.float32),
            pltpu.VMEM((b_per_w,), jnp.float32),
            pltpu.VMEM((b_per_w,), jnp.float32),
            pltpu.VMEM((b_per_w,), jnp.float32),
            [pltpu.SemaphoreType.DMA] * 3,
        ],
    )
    def k2(ctx_hbm, cemb_hbm, cb_hbm, wi_hbm, bi_hbm, out_hbm,
           idx_j, wi_v, wj_v, bi_v, bj_v, out_v, sems):
        wid = lax.axis_index("s") * NC + lax.axis_index("c")
        base = wid * b_per_w
        pltpu.sync_copy(ctx_hbm.at[pl.ds(base, b_per_w)], idx_j)
        for c in range(n_chunks):
            sl = pl.ds(c * CHUNK, CHUNK)
            pltpu.async_copy(cb_hbm.at[idx_j.at[sl]], bj_v.at[sl], sems[1])
        pltpu.sync_copy(bi_hbm.at[pl.ds(base, b_per_w)], bi_v)

        def issue(b, carry):
            _issue_rows(idx_j, cemb_hbm, wj_v, sems[0], b, rows_per_pass)
            return carry

        lane_ids = lax.iota(jnp.int32, LANES)

        def block(b, carry):
            r0 = b * LANES
            dr0 = (b % (rows_per_pass // LANES)) * LANES
            sums = jnp.zeros((LANES,), jnp.float32)
            for r in range(LANES):
                row = dr0 + r
                acc = wi_v[row, pl.ds(0, LANES)] * wj_v[row, pl.ds(0, LANES)]
                for s in range(1, n_seg):
                    sl = pl.ds(s * LANES, LANES)
                    acc = acc + wi_v[row, sl] * wj_v[row, sl]
                sums = jnp.where(lane_ids == r, jnp.sum(acc), sums)
            blk = pl.ds(r0, LANES)
            out_v[blk] = sums + bi_v[blk] + bj_v[blk]
            return carry

        pltpu.make_async_copy(cb_hbm.at[pl.ds(0, b_per_w)], bj_v,
                              sems[1]).wait()
        blocks_per_pass = rows_per_pass // LANES
        for p in range(N_PASS):
            lax.fori_loop(p * blocks_per_pass, (p + 1) * blocks_per_pass,
                          issue, 0)
            pltpu.async_copy(
                wi_hbm.at[pl.ds(base + p * rows_per_pass, rows_per_pass)],
                wi_v, sems[2])
            pltpu.make_async_copy(
                wi_hbm.at[pl.ds(0, rows_per_pass)], wi_v, sems[2]).wait()
            pltpu.make_async_copy(cemb_hbm.at[pl.ds(0, rows_per_pass)],
                                  wj_v, sems[0]).wait()
            lax.fori_loop(p * blocks_per_pass, (p + 1) * blocks_per_pass,
                          block, 0)
        pltpu.sync_copy(out_v, out_hbm.at[pl.ds(base, b_per_w)])

    return k2


def kernel(token, context_token, token_embedding, context_embedding,
           token_bias, context_bias):
    B = token.shape[0]
    D = token_embedding.shape[1]
    tok = token.astype(jnp.int32)
    ctx = context_token.astype(jnp.int32)
    tb = token_bias.sum(axis=1)
    cb = context_bias.sum(axis=1)
    wi_rows, bi = _build_k1(B, D)(tok, token_embedding, tb)
    return _build_k2(B, D)(ctx, context_embedding, cb, wi_rows, bi)
